# Initial kernel scaffold; baseline (speedup 1.0000x reference)
import jax
import jax.numpy as jnp
from jax.experimental import pallas as pl


def kernel(x_batch, edge_index, edge_attr, params):
    raise NotImplementedError("write your pallas kernel here")


# speedup vs baseline: 4.8823x; 4.8823x over previous
import functools

import jax
import jax.numpy as jnp
from jax import lax
from jax.experimental import pallas as pl

N = 50000
E = 800000
D = 64
L = 25
B = 2

NBLK = 2000
EBLK = 8000

_F32 = jnp.float32


def _ln(x, g, b, eps=1e-5):
    mu = jnp.mean(x, axis=-1, keepdims=True)
    var = jnp.mean((x - mu) * (x - mu), axis=-1, keepdims=True)
    return (x - mu) / jnp.sqrt(var + eps) * g + b


def _mm(a, b):
    return jax.lax.dot_general(a, b, (((1,), (0,)), ((), ())),
                               preferred_element_type=_F32)


def _embed_body(x_ref, WeT, be, W1aT, W1bT, h_out, p_out, q_out):
    h = _mm(x_ref[...], WeT[...]) + be[...]
    h_out[...] = h
    p_out[...] = _mm(h, W1aT[...])
    q_out[...] = _mm(h, W1bT[...])


def _embed(x, WeT, be, W1aT0, W1bT0):
    grid = (N // NBLK,)
    bs_n = lambda c: pl.BlockSpec((NBLK, c), lambda i: (i, 0))
    full = lambda a: pl.BlockSpec(a.shape, lambda i: (0,) * a.ndim)
    return pl.pallas_call(
        _embed_body,
        grid=grid,
        in_specs=[bs_n(3), full(WeT), full(be), full(W1aT0), full(W1bT0)],
        out_specs=[bs_n(D), bs_n(D), bs_n(D)],
        out_shape=[jax.ShapeDtypeStruct((N, D), _F32)] * 3,
    )(x, WeT, be, W1aT0, W1bT0)


def _edge_body(gp_ref, gq_ref, ea_ref, W1cT, b1, g1, bb1, mr2_out):
    m = gp_ref[...] + gq_ref[...] + _mm(ea_ref[...], W1cT[...]) + b1[...]
    mr = jax.nn.relu(_ln(m, g1[...], bb1[...]))
    mr2_out[...] = jnp.stack([mr[:, :32], mr[:, 32:]], axis=0)


def _edge_stage(gp, gq, ea, W1cT, b1, g1, bb1):
    grid = (E // EBLK,)
    bs_e = lambda c: pl.BlockSpec((EBLK, c), lambda i: (i, 0))
    full = lambda a: pl.BlockSpec(a.shape, lambda i: (0,) * a.ndim)
    return pl.pallas_call(
        _edge_body,
        grid=grid,
        in_specs=[bs_e(D), bs_e(D), bs_e(2), full(W1cT), full(b1),
                  full(g1), full(bb1)],
        out_specs=pl.BlockSpec((2, EBLK, 32), lambda i: (0, i, 0)),
        out_shape=jax.ShapeDtypeStruct((2, E, 32), _F32),
    )(gp, gq, ea, W1cT, b1, g1, bb1)


def _node_body(h_ref, s2_ref, deg_ref, W2T, b2, W3hT, W3aT, b3, g2, bb2,
               W1aT, W1bT, h_out, p_out, q_out):
    s2 = s2_ref[...]
    S = jnp.concatenate([s2[0], s2[1]], axis=1)
    aggr = _mm(S, W2T[...]) + deg_ref[...] * b2[...]
    h = h_ref[...]
    u = _mm(h, W3hT[...]) + _mm(aggr, W3aT[...]) + b3[...]
    u = jax.nn.relu(_ln(u, g2[...], bb2[...]))
    hn = h + u
    h_out[...] = hn
    p_out[...] = _mm(hn, W1aT[...])
    q_out[...] = _mm(hn, W1bT[...])


def _node_stage(h, s2, deg, W2T, b2, W3hT, W3aT, b3, g2, bb2, W1aT, W1bT):
    grid = (N // NBLK,)
    bs_n = lambda c: pl.BlockSpec((NBLK, c), lambda i: (i, 0))
    full = lambda a: pl.BlockSpec(a.shape, lambda i: (0,) * a.ndim)
    return pl.pallas_call(
        _node_body,
        grid=grid,
        in_specs=[bs_n(D), pl.BlockSpec((2, NBLK, 32), lambda i: (0, i, 0)),
                  bs_n(1), full(W2T), full(b2), full(W3hT), full(W3aT),
                  full(b3), full(g2), full(bb2), full(W1aT), full(W1bT)],
        out_specs=[bs_n(D), bs_n(D), bs_n(D)],
        out_shape=[jax.ShapeDtypeStruct((N, D), _F32)] * 3,
    )(h, s2, deg, W2T, b2, W3hT, W3aT, b3, g2, bb2, W1aT, W1bT)


def _dec_body(h_ref, Wd1T, bd1, Wd2T, bd2, out_ref):
    hid = jax.nn.relu(_mm(h_ref[...], Wd1T[...]) + bd1[...])
    out_ref[...] = _mm(hid, Wd2T[...]) + bd2[...]


def _decode(h, Wd1T, bd1, Wd2T, bd2):
    grid = (N // NBLK,)
    bs_n = lambda c: pl.BlockSpec((NBLK, c), lambda i: (i, 0))
    full = lambda a: pl.BlockSpec(a.shape, lambda i: (0,) * a.ndim)
    return pl.pallas_call(
        _dec_body,
        grid=grid,
        in_specs=[bs_n(D), full(Wd1T), full(bd1), full(Wd2T), full(bd2)],
        out_specs=bs_n(2),
        out_shape=jax.ShapeDtypeStruct((N, 2), _F32),
    )(h, Wd1T, bd1, Wd2T, bd2)


def _gather_pq(p, q, row, col):
    return jnp.take(p, row, axis=0), jnp.take(q, col, axis=0)


def _scatter_s2(mr2, row):
    s_lo = jnp.zeros((N, 32), _F32).at[row].add(mr2[0])
    s_hi = jnp.zeros((N, 32), _F32).at[row].add(mr2[1])
    return jnp.stack([s_lo, s_hi], axis=0)


def kernel(x_batch, edge_index, edge_attr, params):
    row = edge_index[0]
    col = edge_index[1]
    deg = jnp.zeros((N, 1), _F32).at[row, 0].add(1.0)

    W1 = params["W1"]
    W1aT = jnp.swapaxes(W1[:, :, :D], 1, 2)
    W1bT = jnp.swapaxes(W1[:, :, D:2 * D], 1, 2)
    W1cT = jnp.swapaxes(W1[:, :, 2 * D:], 1, 2)
    W2T = jnp.swapaxes(params["W2"], 1, 2)
    W3 = params["W3"]
    W3hT = jnp.swapaxes(W3[:, :, :D], 1, 2)
    W3aT = jnp.swapaxes(W3[:, :, D:], 1, 2)
    W1aT_nxt = jnp.roll(W1aT, -1, axis=0)
    W1bT_nxt = jnp.roll(W1bT, -1, axis=0)
    row2 = lambda a: a.reshape(1, -1)
    WeT = params["We"].T
    be = row2(params["be"])
    Wd1T = params["Wd1"].T
    bd1 = row2(params["bd1"])
    Wd2T = params["Wd2"].T
    bd2 = row2(params["bd2"])

    b1 = params["b1"][:, None, :]
    g1 = params["g1"][:, None, :]
    bb1 = params["bb1"][:, None, :]
    b2 = params["b2"][:, None, :]
    b3 = params["b3"][:, None, :]
    g2 = params["g2"][:, None, :]
    bb2 = params["bb2"][:, None, :]

    layer_xs = (W1cT, b1, g1, bb1, W2T, b2, W3hT, W3aT, b3, g2, bb2,
                W1aT_nxt, W1bT_nxt)

    def single(x):
        h, p, q = _embed(x, WeT, be, W1aT[0], W1bT[0])

        def body(carry, lp):
            h, p, q = carry
            (w1cT, b1l, g1l, bb1l, w2T, b2l, w3hT, w3aT, b3l, g2l, bb2l,
             w1aTn, w1bTn) = lp
            gp, gq = _gather_pq(p, q, row, col)
            mr2 = _edge_stage(gp, gq, edge_attr, w1cT, b1l, g1l, bb1l)
            s2 = _scatter_s2(mr2, row)
            h, p, q = _node_stage(h, s2, deg, w2T, b2l, w3hT, w3aT, b3l,
                                  g2l, bb2l, w1aTn, w1bTn)
            return (h, p, q), None

        (h, _, _), _ = lax.scan(body, (h, p, q), layer_xs)
        return _decode(h, Wd1T, bd1, Wd2T, bd2)

    return jnp.stack([single(x_batch[0]), single(x_batch[1])], axis=0)


# speedup vs baseline: 13.4640x; 2.7577x over previous
import functools

import jax
import jax.numpy as jnp
from jax import lax
from jax.experimental import pallas as pl
from jax.experimental.pallas import tpu as pltpu
from jax.experimental.pallas import tpu_sc as plsc

N = 50000
E = 800000
D = 64
L = 25
B = 2

NBLK = 2000
EBLK = 8192
CH = 128
E_PAD = 802816
ACC_ROWS = 51200

_F32 = jnp.float32


def _ln(x, g, b, eps=1e-5):
    mu = jnp.mean(x, axis=-1, keepdims=True)
    var = jnp.mean((x - mu) * (x - mu), axis=-1, keepdims=True)
    return (x - mu) / jnp.sqrt(var + eps) * g + b


def _mm(a, b):
    return jax.lax.dot_general(a, b, (((1,), (0,)), ((), ())),
                               preferred_element_type=_F32)


def _embed_body(x_ref, WeT, be, W1aT, W1bT, h_out, p_out, q_out):
    h = _mm(x_ref[...], WeT[...]) + be[...]
    h_out[...] = h
    p_out[...] = _mm(h, W1aT[...])
    q_out[...] = _mm(h, W1bT[...])


def _embed(x, WeT, be, W1aT0, W1bT0):
    grid = (N // NBLK,)
    bs_n = lambda c: pl.BlockSpec((NBLK, c), lambda i: (i, 0))
    full = lambda a: pl.BlockSpec(a.shape, lambda i: (0,) * a.ndim)
    return pl.pallas_call(
        _embed_body,
        grid=grid,
        in_specs=[bs_n(3), full(WeT), full(be), full(W1aT0), full(W1bT0)],
        out_specs=[bs_n(D), bs_n(D), bs_n(D)],
        out_shape=[jax.ShapeDtypeStruct((N, D), _F32)] * 3,
    )(x, WeT, be, W1aT0, W1bT0)


def _edge_body(gp_ref, gq_ref, ea_ref, W1cT, b1, g1, bb1, mr2_out):
    m = gp_ref[...] + gq_ref[...] + _mm(ea_ref[...], W1cT[...]) + b1[...]
    mr = jax.nn.relu(_ln(m, g1[...], bb1[...]))
    mr2_out[...] = jnp.stack([mr[:, :32], mr[:, 32:]], axis=0)


def _edge_stage(gp, gq, ea, W1cT, b1, g1, bb1):
    grid = (E_PAD // EBLK,)
    bs_e = lambda c: pl.BlockSpec((EBLK, c), lambda i: (i, 0))
    full = lambda a: pl.BlockSpec(a.shape, lambda i: (0,) * a.ndim)
    return pl.pallas_call(
        _edge_body,
        grid=grid,
        in_specs=[bs_e(D), bs_e(D), bs_e(2), full(W1cT), full(b1),
                  full(g1), full(bb1)],
        out_specs=pl.BlockSpec((2, EBLK, 32), lambda i: (0, i, 0)),
        out_shape=jax.ShapeDtypeStruct((2, E_PAD, 32), _F32),
    )(gp, gq, ea, W1cT, b1, g1, bb1)


def _node_body(h_ref, s2_ref, deg_ref, W2T, b2, W3hT, W3aT, b3, g2, bb2,
               W1aT, W1bT, h_out, p_out, q_out):
    s2 = s2_ref[...]
    S = jnp.concatenate([s2[0], s2[1]], axis=1)
    aggr = _mm(S, W2T[...]) + deg_ref[...] * b2[...]
    h = h_ref[...]
    u = _mm(h, W3hT[...]) + _mm(aggr, W3aT[...]) + b3[...]
    u = jax.nn.relu(_ln(u, g2[...], bb2[...]))
    hn = h + u
    h_out[...] = hn
    p_out[...] = _mm(hn, W1aT[...])
    q_out[...] = _mm(hn, W1bT[...])


def _node_stage(h, s2, deg, W2T, b2, W3hT, W3aT, b3, g2, bb2, W1aT, W1bT):
    grid = (N // NBLK,)
    bs_n = lambda c: pl.BlockSpec((NBLK, c), lambda i: (i, 0))
    full = lambda a: pl.BlockSpec(a.shape, lambda i: (0,) * a.ndim)
    return pl.pallas_call(
        _node_body,
        grid=grid,
        in_specs=[bs_n(D), pl.BlockSpec((2, NBLK, 32), lambda i: (0, i, 0)),
                  bs_n(1), full(W2T), full(b2), full(W3hT), full(W3aT),
                  full(b3), full(g2), full(bb2), full(W1aT), full(W1bT)],
        out_specs=[bs_n(D), bs_n(D), bs_n(D)],
        out_shape=[jax.ShapeDtypeStruct((N, D), _F32)] * 3,
    )(h, s2, deg, W2T, b2, W3hT, W3aT, b3, g2, bb2, W1aT, W1bT)


def _dec_body(h_ref, Wd1T, bd1, Wd2T, bd2, out_ref):
    hid = jax.nn.relu(_mm(h_ref[...], Wd1T[...]) + bd1[...])
    out_ref[...] = _mm(hid, Wd2T[...]) + bd2[...]


def _decode(h, Wd1T, bd1, Wd2T, bd2):
    grid = (N // NBLK,)
    bs_n = lambda c: pl.BlockSpec((NBLK, c), lambda i: (i, 0))
    full = lambda a: pl.BlockSpec(a.shape, lambda i: (0,) * a.ndim)
    return pl.pallas_call(
        _dec_body,
        grid=grid,
        in_specs=[bs_n(D), full(Wd1T), full(bd1), full(Wd2T), full(bd2)],
        out_specs=bs_n(2),
        out_shape=jax.ShapeDtypeStruct((N, 2), _F32),
    )(h, Wd1T, bd1, Wd2T, bd2)


_SC_MESH = plsc.VectorSubcoreMesh(core_axis_name="c", subcore_axis_name="s",
                                  num_cores=2, num_subcores=16)
_SC_PARAMS = pltpu.CompilerParams(use_tc_tiling_on_sc=False)

_G_CHUNKS = E_PAD // (32 * CH)
_S_CHUNKS = E_PAD // (16 * CH)


def _sc_gather_body(p_hbm, q_hbm, row_hbm, col_hbm, gp_hbm, gq_hbm,
                    idx_r, idx_c, bufp, bufq, sem1, sem2):
    wid = lax.axis_index("s") * 2 + lax.axis_index("c")

    def step(j, carry):
        base = (wid * _G_CHUNKS + j) * CH
        pltpu.sync_copy(row_hbm.at[pl.ds(base, CH)], idx_r)
        pltpu.sync_copy(col_hbm.at[pl.ds(base, CH)], idx_c)
        c1 = pltpu.async_copy(p_hbm.at[idx_r], bufp, sem1)
        c2 = pltpu.async_copy(q_hbm.at[idx_c], bufq, sem2)
        c1.wait()
        c2.wait()
        pltpu.sync_copy(bufp, gp_hbm.at[pl.ds(base, CH)])
        pltpu.sync_copy(bufq, gq_hbm.at[pl.ds(base, CH)])
        return carry

    lax.fori_loop(0, _G_CHUNKS, step, 0)


@functools.partial(
    pl.kernel,
    out_type=[jax.ShapeDtypeStruct((E_PAD, D), _F32)] * 2,
    mesh=_SC_MESH,
    scratch_types=[
        pltpu.VMEM((CH,), jnp.int32),
        pltpu.VMEM((CH,), jnp.int32),
        pltpu.VMEM((CH, D), _F32),
        pltpu.VMEM((CH, D), _F32),
        pltpu.SemaphoreType.DMA,
        pltpu.SemaphoreType.DMA,
    ],
    compiler_params=_SC_PARAMS,
)
def _sc_gather(p_hbm, q_hbm, row_hbm, col_hbm, gp_hbm, gq_hbm,
               idx_r, idx_c, bufp, bufq, sem1, sem2):
    _sc_gather_body(p_hbm, q_hbm, row_hbm, col_hbm, gp_hbm, gq_hbm,
                    idx_r, idx_c, bufp, bufq, sem1, sem2)


_WRITE_CHUNK = 625


@functools.partial(
    pl.kernel,
    out_type=jax.ShapeDtypeStruct((2, N, 32), _F32),
    mesh=_SC_MESH,
    scratch_types=[
        pltpu.VMEM_SHARED((ACC_ROWS, 32), _F32),
        pltpu.VMEM((CH,), jnp.int32),
        pltpu.VMEM((CH, 32), _F32),
        pltpu.VMEM((CH, 32), _F32),
        pltpu.VMEM((_WRITE_CHUNK, 32), _F32),
    ],
    compiler_params=_SC_PARAMS,
)
def _sc_scatter(mr2_hbm, rows_hbm, zeros_hbm, s2_hbm,
                acc, idx_v, mrbuf, zbuf, stage):
    c = lax.axis_index("c")
    t = lax.axis_index("s")

    pltpu.sync_copy(zeros_hbm, zbuf)

    def zstep(j, carry):
        pltpu.sync_copy(zbuf, acc.at[pl.ds((t * 25 + j) * CH, CH)])
        return carry

    lax.fori_loop(0, ACC_ROWS // (16 * CH), zstep, 0)
    plsc.subcore_barrier()

    def sstep(j, carry):
        base = (t * _S_CHUNKS + j) * CH
        pltpu.sync_copy(rows_hbm.at[pl.ds(base, CH)], idx_v)
        pltpu.sync_copy(mr2_hbm.at[c, pl.ds(base, CH)], mrbuf)
        pltpu.sync_copy(mrbuf, acc.at[idx_v], add=True)
        return carry

    lax.fori_loop(0, _S_CHUNKS, sstep, 0)
    plsc.subcore_barrier()

    def wstep(k, carry):
        off = t * (N // 16) + k * _WRITE_CHUNK
        pltpu.sync_copy(acc.at[pl.ds(off, _WRITE_CHUNK)], stage)
        pltpu.sync_copy(stage, s2_hbm.at[c, pl.ds(off, _WRITE_CHUNK)])
        return carry

    lax.fori_loop(0, (N // 16) // _WRITE_CHUNK, wstep, 0)


def kernel(x_batch, edge_index, edge_attr, params):
    row = edge_index[0]
    col = edge_index[1]
    deg = jnp.zeros((N, 1), _F32).at[row, 0].add(1.0)

    pad = E_PAD - E
    row_g = jnp.concatenate([row, jnp.zeros((pad,), jnp.int32)])
    col_g = jnp.concatenate([col, jnp.zeros((pad,), jnp.int32)])
    row_s = jnp.concatenate([row, jnp.full((pad,), N, jnp.int32)])
    ea_pad = jnp.concatenate([edge_attr, jnp.zeros((pad, 2), _F32)])
    zeros128 = jnp.zeros((CH, 32), _F32)

    W1 = params["W1"]
    W1aT = jnp.swapaxes(W1[:, :, :D], 1, 2)
    W1bT = jnp.swapaxes(W1[:, :, D:2 * D], 1, 2)
    W1cT = jnp.swapaxes(W1[:, :, 2 * D:], 1, 2)
    W2T = jnp.swapaxes(params["W2"], 1, 2)
    W3 = params["W3"]
    W3hT = jnp.swapaxes(W3[:, :, :D], 1, 2)
    W3aT = jnp.swapaxes(W3[:, :, D:], 1, 2)
    W1aT_nxt = jnp.roll(W1aT, -1, axis=0)
    W1bT_nxt = jnp.roll(W1bT, -1, axis=0)
    row2 = lambda a: a.reshape(1, -1)
    WeT = params["We"].T
    be = row2(params["be"])
    Wd1T = params["Wd1"].T
    bd1 = row2(params["bd1"])
    Wd2T = params["Wd2"].T
    bd2 = row2(params["bd2"])

    b1 = params["b1"][:, None, :]
    g1 = params["g1"][:, None, :]
    bb1 = params["bb1"][:, None, :]
    b2 = params["b2"][:, None, :]
    b3 = params["b3"][:, None, :]
    g2 = params["g2"][:, None, :]
    bb2 = params["bb2"][:, None, :]

    layer_xs = (W1cT, b1, g1, bb1, W2T, b2, W3hT, W3aT, b3, g2, bb2,
                W1aT_nxt, W1bT_nxt)

    def single(x):
        h, p, q = _embed(x, WeT, be, W1aT[0], W1bT[0])

        def body(carry, lp):
            h, p, q = carry
            (w1cT, b1l, g1l, bb1l, w2T, b2l, w3hT, w3aT, b3l, g2l, bb2l,
             w1aTn, w1bTn) = lp
            gp, gq = _sc_gather(p, q, row_g, col_g)
            mr2 = _edge_stage(gp, gq, ea_pad, w1cT, b1l, g1l, bb1l)
            s2 = _sc_scatter(mr2, row_s, zeros128)
            h, p, q = _node_stage(h, s2, deg, w2T, b2l, w3hT, w3aT, b3l,
                                  g2l, bb2l, w1aTn, w1bTn)
            return (h, p, q), None

        (h, _, _), _ = lax.scan(body, (h, p, q), layer_xs)
        return _decode(h, Wd1T, bd1, Wd2T, bd2)

    return jnp.stack([single(x_batch[0]), single(x_batch[1])], axis=0)


# speedup vs baseline: 16.5448x; 1.2288x over previous
import functools

import jax
import jax.numpy as jnp
from jax import lax
from jax.experimental import pallas as pl
from jax.experimental.pallas import tpu as pltpu
from jax.experimental.pallas import tpu_sc as plsc

N = 50000
E = 800000
D = 64
L = 25
B = 2

NBLK = 2000
EBLK = 8192
CH = 128
E_PAD = 802816
ACC_ROWS = 51200

_F32 = jnp.float32


def _ln(x, g, b, eps=1e-5):
    mu = jnp.mean(x, axis=-1, keepdims=True)
    var = jnp.mean((x - mu) * (x - mu), axis=-1, keepdims=True)
    return (x - mu) / jnp.sqrt(var + eps) * g + b


def _mm(a, b):
    return jax.lax.dot_general(a, b, (((1,), (0,)), ((), ())),
                               preferred_element_type=_F32)


def _embed_body(x_ref, WeT, be, W1aT, W1bT, h_out, p_out, q_out):
    h = _mm(x_ref[...], WeT[...]) + be[...]
    h_out[...] = h
    p_out[...] = _mm(h, W1aT[...])
    q_out[...] = _mm(h, W1bT[...])


def _embed(x, WeT, be, W1aT0, W1bT0):
    grid = (N // NBLK,)
    bs_n = lambda c: pl.BlockSpec((NBLK, c), lambda i: (i, 0))
    full = lambda a: pl.BlockSpec(a.shape, lambda i: (0,) * a.ndim)
    return pl.pallas_call(
        _embed_body,
        grid=grid,
        in_specs=[bs_n(3), full(WeT), full(be), full(W1aT0), full(W1bT0)],
        out_specs=[bs_n(D), bs_n(D), bs_n(D)],
        out_shape=[jax.ShapeDtypeStruct((N, D), _F32)] * 3,
    )(x, WeT, be, W1aT0, W1bT0)


def _edge_body(gp_ref, gq_ref, ea_ref, W1cT, b1, g1, bb1, mr2_out):
    m = gp_ref[...] + gq_ref[...] + _mm(ea_ref[...], W1cT[...]) + b1[...]
    mr = jax.nn.relu(_ln(m, g1[...], bb1[...]))
    mr2_out[...] = jnp.stack([mr[:, :32], mr[:, 32:]], axis=0)


def _edge_stage(gp, gq, ea, W1cT, b1, g1, bb1):
    grid = (E_PAD // EBLK,)
    bs_e = lambda c: pl.BlockSpec((EBLK, c), lambda i: (i, 0))
    full = lambda a: pl.BlockSpec(a.shape, lambda i: (0,) * a.ndim)
    return pl.pallas_call(
        _edge_body,
        grid=grid,
        in_specs=[bs_e(D), bs_e(D), bs_e(2), full(W1cT), full(b1),
                  full(g1), full(bb1)],
        out_specs=pl.BlockSpec((2, EBLK, 32), lambda i: (0, i, 0)),
        out_shape=jax.ShapeDtypeStruct((2, E_PAD, 32), _F32),
    )(gp, gq, ea, W1cT, b1, g1, bb1)


def _node_body(h_ref, s2_ref, deg_ref, W2T, b2, W3hT, W3aT, b3, g2, bb2,
               W1aT, W1bT, h_out, p_out, q_out):
    s2 = s2_ref[...]
    S = jnp.concatenate([s2[0], s2[1]], axis=1)
    aggr = _mm(S, W2T[...]) + deg_ref[...] * b2[...]
    h = h_ref[...]
    u = _mm(h, W3hT[...]) + _mm(aggr, W3aT[...]) + b3[...]
    u = jax.nn.relu(_ln(u, g2[...], bb2[...]))
    hn = h + u
    h_out[...] = hn
    p_out[...] = _mm(hn, W1aT[...])
    q_out[...] = _mm(hn, W1bT[...])


def _node_stage(h, s2, deg, W2T, b2, W3hT, W3aT, b3, g2, bb2, W1aT, W1bT):
    grid = (N // NBLK,)
    bs_n = lambda c: pl.BlockSpec((NBLK, c), lambda i: (i, 0))
    full = lambda a: pl.BlockSpec(a.shape, lambda i: (0,) * a.ndim)
    return pl.pallas_call(
        _node_body,
        grid=grid,
        in_specs=[bs_n(D), pl.BlockSpec((2, NBLK, 32), lambda i: (0, i, 0)),
                  bs_n(1), full(W2T), full(b2), full(W3hT), full(W3aT),
                  full(b3), full(g2), full(bb2), full(W1aT), full(W1bT)],
        out_specs=[bs_n(D), bs_n(D), bs_n(D)],
        out_shape=[jax.ShapeDtypeStruct((N, D), _F32)] * 3,
    )(h, s2, deg, W2T, b2, W3hT, W3aT, b3, g2, bb2, W1aT, W1bT)


def _dec_body(h_ref, Wd1T, bd1, Wd2T, bd2, out_ref):
    hid = jax.nn.relu(_mm(h_ref[...], Wd1T[...]) + bd1[...])
    out_ref[...] = _mm(hid, Wd2T[...]) + bd2[...]


def _decode(h, Wd1T, bd1, Wd2T, bd2):
    grid = (N // NBLK,)
    bs_n = lambda c: pl.BlockSpec((NBLK, c), lambda i: (i, 0))
    full = lambda a: pl.BlockSpec(a.shape, lambda i: (0,) * a.ndim)
    return pl.pallas_call(
        _dec_body,
        grid=grid,
        in_specs=[bs_n(D), full(Wd1T), full(bd1), full(Wd2T), full(bd2)],
        out_specs=bs_n(2),
        out_shape=jax.ShapeDtypeStruct((N, 2), _F32),
    )(h, Wd1T, bd1, Wd2T, bd2)


_SC_MESH = plsc.VectorSubcoreMesh(core_axis_name="c", subcore_axis_name="s",
                                  num_cores=2, num_subcores=16)
_SC_PARAMS = pltpu.CompilerParams(use_tc_tiling_on_sc=False)

_E_TILE = E_PAD // 32
_GC = 448
_G_CHUNKS = _E_TILE // _GC
_S_CHUNKS = E_PAD // (16 * CH)
_SRD = 256
_S_OUTER = E_PAD // (16 * _SRD)


def _gather_pass(tab_hbm, idx_hbm, out_hbm, idx_all, buf0, buf1, sem0, sem1,
                 tbase):
    pltpu.sync_copy(idx_hbm.at[pl.ds(tbase, _E_TILE)], idx_all)

    def _g(j, buf, sem):
        return pltpu.async_copy(
            tab_hbm.at[idx_all.at[pl.ds(j * _GC, _GC)]], buf, sem)

    _g(0, buf0, sem0)
    _g(1, buf1, sem1)

    def step(k, carry):
        for b, (buf, sem) in enumerate(((buf0, sem0), (buf1, sem1))):
            j = 2 * k + b
            pltpu.make_async_copy(
                tab_hbm.at[idx_all.at[pl.ds(j * _GC, _GC)]], buf, sem).wait()
            pltpu.sync_copy(buf, out_hbm.at[pl.ds(tbase + j * _GC, _GC)])

            @pl.when(j + 2 < _G_CHUNKS)
            def _():
                _g(j + 2, buf, sem)
        return carry

    lax.fori_loop(0, _G_CHUNKS // 2, step, 0)


@functools.partial(
    pl.kernel,
    out_type=[jax.ShapeDtypeStruct((E_PAD, D), _F32)] * 2,
    mesh=_SC_MESH,
    scratch_types=[
        pltpu.VMEM((_E_TILE,), jnp.int32),
        pltpu.VMEM((_GC, D), _F32),
        pltpu.VMEM((_GC, D), _F32),
        pltpu.SemaphoreType.DMA,
        pltpu.SemaphoreType.DMA,
    ],
    compiler_params=_SC_PARAMS,
)
def _sc_gather(p_hbm, q_hbm, row_hbm, col_hbm, gp_hbm, gq_hbm,
               idx_all, buf0, buf1, sem0, sem1):
    wid = lax.axis_index("s") * 2 + lax.axis_index("c")
    tbase = wid * _E_TILE
    _gather_pass(p_hbm, row_hbm, gp_hbm, idx_all, buf0, buf1, sem0, sem1,
                 tbase)
    _gather_pass(q_hbm, col_hbm, gq_hbm, idx_all, buf0, buf1, sem0, sem1,
                 tbase)


_WRITE_CHUNK = 625


@functools.partial(
    pl.kernel,
    out_type=jax.ShapeDtypeStruct((2, N, 32), _F32),
    mesh=_SC_MESH,
    scratch_types=[
        pltpu.VMEM_SHARED((ACC_ROWS, 32), _F32),
        pltpu.VMEM((_SRD // CH, CH), jnp.int32),
        pltpu.VMEM((_SRD // CH, CH), jnp.int32),
        pltpu.VMEM((_SRD, 32), _F32),
        pltpu.VMEM((_SRD, 32), _F32),
        pltpu.SemaphoreType.DMA,
        pltpu.SemaphoreType.DMA,
        pltpu.SemaphoreType.DMA,
        pltpu.SemaphoreType.DMA,
        pltpu.SemaphoreType.DMA,
        pltpu.SemaphoreType.DMA,
    ],
    compiler_params=_SC_PARAMS,
)
def _sc_scatter(mr2_hbm, rows_hbm, zeros_hbm, s2_hbm,
                acc, idxr0, idxr1, mrbuf0, mrbuf1,
                semr0, semr1, sema0, sema1, semi0, semi1):
    c = lax.axis_index("c")
    t = lax.axis_index("s")
    tbase = t * _S_OUTER * _SRD
    _SUB = _SRD // CH

    pltpu.sync_copy(zeros_hbm, mrbuf0.at[pl.ds(0, CH)])

    def zstep(j, carry):
        pltpu.sync_copy(mrbuf0.at[pl.ds(0, CH)],
                        acc.at[pl.ds((t * 25 + j) * CH, CH)])
        return carry

    lax.fori_loop(0, ACC_ROWS // (16 * CH), zstep, 0)
    plsc.subcore_barrier()

    def _rd(o, buf, sem):
        return pltpu.async_copy(
            mr2_hbm.at[c, pl.ds(tbase + o * _SRD, _SRD)], buf, sem)

    def _rdidx(o, idxr, semi):
        for s in range(_SUB):
            pltpu.async_copy(
                rows_hbm.at[pl.ds(tbase + o * _SRD + s * CH, CH)],
                idxr.at[s], semi)

    def _widx(o, idxr, semi):
        for s in range(_SUB):
            pltpu.make_async_copy(
                rows_hbm.at[pl.ds(tbase + o * _SRD + s * CH, CH)],
                idxr.at[s], semi).wait()

    _rd(0, mrbuf0, semr0)
    _rd(1, mrbuf1, semr1)
    _rdidx(0, idxr0, semi0)
    _rdidx(1, idxr1, semi1)

    def sstep(k, carry):
        for b, (buf, idxr, semr, sema, semi) in enumerate(
                ((mrbuf0, idxr0, semr0, sema0, semi0),
                 (mrbuf1, idxr1, semr1, sema1, semi1))):
            o = 2 * k + b
            _widx(o, idxr, semi)
            pltpu.make_async_copy(
                mr2_hbm.at[c, pl.ds(tbase + o * _SRD, _SRD)], buf, semr).wait()
            for s in range(_SUB):
                pltpu.async_copy(
                    buf.at[pl.ds(s * CH, CH)],
                    acc.at[idxr.at[s]], sema, add=True)
            for s in range(_SUB):
                pltpu.make_async_copy(
                    buf.at[pl.ds(s * CH, CH)],
                    acc.at[idxr.at[s]], sema).wait()

            @pl.when(o + 2 < _S_OUTER)
            def _():
                _rd(o + 2, buf, semr)
                _rdidx(o + 2, idxr, semi)
        return carry

    lax.fori_loop(0, _S_OUTER // 2, sstep, 0)
    plsc.subcore_barrier()

    def wstep(k, carry):
        off = t * (N // 16) + k * 125
        pltpu.sync_copy(acc.at[pl.ds(off, 125)], mrbuf0.at[pl.ds(0, 125)])
        pltpu.sync_copy(mrbuf0.at[pl.ds(0, 125)], s2_hbm.at[c, pl.ds(off, 125)])
        return carry

    lax.fori_loop(0, (N // 16) // 125, wstep, 0)


def kernel(x_batch, edge_index, edge_attr, params):
    row = edge_index[0]
    col = edge_index[1]
    deg = jnp.zeros((N, 1), _F32).at[row, 0].add(1.0)

    pad = E_PAD - E
    row_g = jnp.concatenate([row, jnp.zeros((pad,), jnp.int32)])
    col_g = jnp.concatenate([col, jnp.zeros((pad,), jnp.int32)])
    row_s = jnp.concatenate([row, jnp.full((pad,), N, jnp.int32)])
    ea_pad = jnp.concatenate([edge_attr, jnp.zeros((pad, 2), _F32)])
    zeros128 = jnp.zeros((CH, 32), _F32)

    W1 = params["W1"]
    W1aT = jnp.swapaxes(W1[:, :, :D], 1, 2)
    W1bT = jnp.swapaxes(W1[:, :, D:2 * D], 1, 2)
    W1cT = jnp.swapaxes(W1[:, :, 2 * D:], 1, 2)
    W2T = jnp.swapaxes(params["W2"], 1, 2)
    W3 = params["W3"]
    W3hT = jnp.swapaxes(W3[:, :, :D], 1, 2)
    W3aT = jnp.swapaxes(W3[:, :, D:], 1, 2)
    W1aT_nxt = jnp.roll(W1aT, -1, axis=0)
    W1bT_nxt = jnp.roll(W1bT, -1, axis=0)
    row2 = lambda a: a.reshape(1, -1)
    WeT = params["We"].T
    be = row2(params["be"])
    Wd1T = params["Wd1"].T
    bd1 = row2(params["bd1"])
    Wd2T = params["Wd2"].T
    bd2 = row2(params["bd2"])

    b1 = params["b1"][:, None, :]
    g1 = params["g1"][:, None, :]
    bb1 = params["bb1"][:, None, :]
    b2 = params["b2"][:, None, :]
    b3 = params["b3"][:, None, :]
    g2 = params["g2"][:, None, :]
    bb2 = params["bb2"][:, None, :]

    layer_xs = (W1cT, b1, g1, bb1, W2T, b2, W3hT, W3aT, b3, g2, bb2,
                W1aT_nxt, W1bT_nxt)

    def single(x):
        h, p, q = _embed(x, WeT, be, W1aT[0], W1bT[0])

        def body(carry, lp):
            h, p, q = carry
            (w1cT, b1l, g1l, bb1l, w2T, b2l, w3hT, w3aT, b3l, g2l, bb2l,
             w1aTn, w1bTn) = lp
            gp, gq = _sc_gather(p, q, row_g, col_g)
            mr2 = _edge_stage(gp, gq, ea_pad, w1cT, b1l, g1l, bb1l)
            s2 = _sc_scatter(mr2, row_s, zeros128)
            h, p, q = _node_stage(h, s2, deg, w2T, b2l, w3hT, w3aT, b3l,
                                  g2l, bb2l, w1aTn, w1bTn)
            return (h, p, q), None

        (h, _, _), _ = lax.scan(body, (h, p, q), layer_xs)
        return _decode(h, Wd1T, bd1, Wd2T, bd2)

    return jnp.stack([single(x_batch[0]), single(x_batch[1])], axis=0)


# speedup vs baseline: 17.4917x; 1.0572x over previous
import functools

import jax
import jax.numpy as jnp
from jax import lax
from jax.experimental import pallas as pl
from jax.experimental.pallas import tpu as pltpu
from jax.experimental.pallas import tpu_sc as plsc

N = 50000
E = 800000
D = 64
L = 25
B = 2

NBLK = 2000
EBLK = 8192
CH = 128
E_PAD = 802816
ACC_ROWS = 51200

_F32 = jnp.float32


def _ln(x, g, b, eps=1e-5):
    mu = jnp.mean(x, axis=-1, keepdims=True)
    var = jnp.mean((x - mu) * (x - mu), axis=-1, keepdims=True)
    return (x - mu) / jnp.sqrt(var + eps) * g + b


def _mm(a, b):
    return jax.lax.dot_general(a, b, (((1,), (0,)), ((), ())),
                               preferred_element_type=_F32)


def _embed_body(x_ref, WeT, be, W1aT, W1bT, h_out, pq_out):
    h = _mm(x_ref[...], WeT[...]) + be[...]
    h_out[...] = h
    pq_out[...] = jnp.concatenate([_mm(h, W1aT[...]), _mm(h, W1bT[...])],
                                  axis=1)


def _embed(x, WeT, be, W1aT0, W1bT0):
    grid = (N // NBLK,)
    bs_n = lambda c: pl.BlockSpec((NBLK, c), lambda i: (i, 0))
    full = lambda a: pl.BlockSpec(a.shape, lambda i: (0,) * a.ndim)
    return pl.pallas_call(
        _embed_body,
        grid=grid,
        in_specs=[bs_n(3), full(WeT), full(be), full(W1aT0), full(W1bT0)],
        out_specs=[bs_n(D), bs_n(2 * D)],
        out_shape=[jax.ShapeDtypeStruct((N, D), _F32),
                   jax.ShapeDtypeStruct((N, 2 * D), _F32)],
    )(x, WeT, be, W1aT0, W1bT0)


def _edge_body(gr_ref, gc_ref, ea_ref, W1cT, b1, g1, bb1, mr2_out):
    m = (gr_ref[...][:, :D] + gc_ref[...][:, D:]
         + _mm(ea_ref[...], W1cT[...]) + b1[...])
    mr = jax.nn.relu(_ln(m, g1[...], bb1[...]))
    mr2_out[...] = jnp.stack([mr[:, :32], mr[:, 32:]], axis=0)


def _edge_stage(gr, gc, ea, W1cT, b1, g1, bb1):
    grid = (E_PAD // EBLK,)
    bs_e = lambda c: pl.BlockSpec((EBLK, c), lambda i: (i, 0))
    full = lambda a: pl.BlockSpec(a.shape, lambda i: (0,) * a.ndim)
    return pl.pallas_call(
        _edge_body,
        grid=grid,
        in_specs=[bs_e(2 * D), bs_e(2 * D), bs_e(2), full(W1cT), full(b1),
                  full(g1), full(bb1)],
        out_specs=pl.BlockSpec((2, EBLK, 32), lambda i: (0, i, 0)),
        out_shape=jax.ShapeDtypeStruct((2, E_PAD, 32), _F32),
    )(gr, gc, ea, W1cT, b1, g1, bb1)


def _node_body(h_ref, s2_ref, deg_ref, W2T, b2, W3hT, W3aT, b3, g2, bb2,
               W1aT, W1bT, h_out, pq_out):
    s2 = s2_ref[...]
    S = jnp.concatenate([s2[0], s2[1]], axis=1)
    aggr = _mm(S, W2T[...]) + deg_ref[...] * b2[...]
    h = h_ref[...]
    u = _mm(h, W3hT[...]) + _mm(aggr, W3aT[...]) + b3[...]
    u = jax.nn.relu(_ln(u, g2[...], bb2[...]))
    hn = h + u
    h_out[...] = hn
    pq_out[...] = jnp.concatenate([_mm(hn, W1aT[...]), _mm(hn, W1bT[...])],
                                  axis=1)


def _node_stage(h, s2, deg, W2T, b2, W3hT, W3aT, b3, g2, bb2, W1aT, W1bT):
    grid = (N // NBLK,)
    bs_n = lambda c: pl.BlockSpec((NBLK, c), lambda i: (i, 0))
    full = lambda a: pl.BlockSpec(a.shape, lambda i: (0,) * a.ndim)
    return pl.pallas_call(
        _node_body,
        grid=grid,
        in_specs=[bs_n(D), pl.BlockSpec((2, NBLK, 32), lambda i: (0, i, 0)),
                  bs_n(1), full(W2T), full(b2), full(W3hT), full(W3aT),
                  full(b3), full(g2), full(bb2), full(W1aT), full(W1bT)],
        out_specs=[bs_n(D), bs_n(2 * D)],
        out_shape=[jax.ShapeDtypeStruct((N, D), _F32),
                   jax.ShapeDtypeStruct((N, 2 * D), _F32)],
    )(h, s2, deg, W2T, b2, W3hT, W3aT, b3, g2, bb2, W1aT, W1bT)


def _dec_body(h_ref, Wd1T, bd1, Wd2T, bd2, out_ref):
    hid = jax.nn.relu(_mm(h_ref[...], Wd1T[...]) + bd1[...])
    out_ref[...] = _mm(hid, Wd2T[...]) + bd2[...]


def _decode(h, Wd1T, bd1, Wd2T, bd2):
    grid = (N // NBLK,)
    bs_n = lambda c: pl.BlockSpec((NBLK, c), lambda i: (i, 0))
    full = lambda a: pl.BlockSpec(a.shape, lambda i: (0,) * a.ndim)
    return pl.pallas_call(
        _dec_body,
        grid=grid,
        in_specs=[bs_n(D), full(Wd1T), full(bd1), full(Wd2T), full(bd2)],
        out_specs=bs_n(2),
        out_shape=jax.ShapeDtypeStruct((N, 2), _F32),
    )(h, Wd1T, bd1, Wd2T, bd2)


_SC_MESH = plsc.VectorSubcoreMesh(core_axis_name="c", subcore_axis_name="s",
                                  num_cores=2, num_subcores=16)
_SC_PARAMS_TILED = pltpu.CompilerParams(use_tc_tiling_on_sc=True)
_SC_PARAMS = pltpu.CompilerParams(use_tc_tiling_on_sc=False)

_E_TILE = E_PAD // 32
_GC = 392
_G_CHUNKS = _E_TILE // _GC
_S_CHUNKS = E_PAD // (16 * CH)
_SRD = 256
_S_OUTER = E_PAD // (16 * _SRD)


def _gather_pass(tab_hbm, idx_hbm, out_hbm, idx_all, buf0, buf1, sem0, sem1,
                 tbase):
    pltpu.sync_copy(idx_hbm.at[pl.ds(tbase, _E_TILE)], idx_all)

    def _g(j, buf, sem):
        return pltpu.async_copy(
            tab_hbm.at[idx_all.at[pl.ds(j * _GC, _GC)]], buf, sem)

    _g(0, buf0, sem0)
    _g(1, buf1, sem1)

    def step(k, carry):
        for b, (buf, sem) in enumerate(((buf0, sem0), (buf1, sem1))):
            j = 2 * k + b
            pltpu.make_async_copy(
                tab_hbm.at[idx_all.at[pl.ds(j * _GC, _GC)]], buf, sem).wait()
            pltpu.sync_copy(buf, out_hbm.at[pl.ds(tbase + j * _GC, _GC)])

            @pl.when(j + 2 < _G_CHUNKS)
            def _():
                _g(j + 2, buf, sem)
        return carry

    lax.fori_loop(0, _G_CHUNKS // 2, step, 0)


@functools.partial(
    pl.kernel,
    out_type=[jax.ShapeDtypeStruct((E_PAD, 2 * D), _F32)] * 2,
    mesh=_SC_MESH,
    scratch_types=[
        pltpu.VMEM((_E_TILE,), jnp.int32),
        pltpu.VMEM((_GC, 2 * D), _F32),
        pltpu.VMEM((_GC, 2 * D), _F32),
        pltpu.SemaphoreType.DMA,
        pltpu.SemaphoreType.DMA,
    ],
    compiler_params=_SC_PARAMS_TILED,
)
def _sc_gather(pq_hbm, row_hbm, col_hbm, gr_hbm, gc_hbm,
               idx_all, buf0, buf1, sem0, sem1):
    wid = lax.axis_index("s") * 2 + lax.axis_index("c")
    tbase = wid * _E_TILE
    _gather_pass(pq_hbm, row_hbm, gr_hbm, idx_all, buf0, buf1, sem0, sem1,
                 tbase)
    _gather_pass(pq_hbm, col_hbm, gc_hbm, idx_all, buf0, buf1, sem0, sem1,
                 tbase)


_WRITE_CHUNK = 625


@functools.partial(
    pl.kernel,
    out_type=jax.ShapeDtypeStruct((2, N, 32), _F32),
    mesh=_SC_MESH,
    scratch_types=[
        pltpu.VMEM_SHARED((ACC_ROWS, 32), _F32),
        pltpu.VMEM((_SRD // CH, CH), jnp.int32),
        pltpu.VMEM((_SRD // CH, CH), jnp.int32),
        pltpu.VMEM((_SRD, 32), _F32),
        pltpu.VMEM((_SRD, 32), _F32),
        pltpu.SemaphoreType.DMA,
        pltpu.SemaphoreType.DMA,
        pltpu.SemaphoreType.DMA,
        pltpu.SemaphoreType.DMA,
        pltpu.SemaphoreType.DMA,
        pltpu.SemaphoreType.DMA,
    ],
    compiler_params=_SC_PARAMS,
)
def _sc_scatter(mr2_hbm, rows_hbm, zeros_hbm, s2_hbm,
                acc, idxr0, idxr1, mrbuf0, mrbuf1,
                semr0, semr1, sema0, sema1, semi0, semi1):
    c = lax.axis_index("c")
    t = lax.axis_index("s")
    tbase = t * _S_OUTER * _SRD
    _SUB = _SRD // CH

    pltpu.sync_copy(zeros_hbm, mrbuf0.at[pl.ds(0, CH)])

    def zstep(j, carry):
        pltpu.sync_copy(mrbuf0.at[pl.ds(0, CH)],
                        acc.at[pl.ds((t * 25 + j) * CH, CH)])
        return carry

    lax.fori_loop(0, ACC_ROWS // (16 * CH), zstep, 0)
    plsc.subcore_barrier()

    def _rd(o, buf, sem):
        return pltpu.async_copy(
            mr2_hbm.at[c, pl.ds(tbase + o * _SRD, _SRD)], buf, sem)

    def _rdidx(o, idxr, semi):
        for s in range(_SUB):
            pltpu.async_copy(
                rows_hbm.at[pl.ds(tbase + o * _SRD + s * CH, CH)],
                idxr.at[s], semi)

    def _widx(o, idxr, semi):
        for s in range(_SUB):
            pltpu.make_async_copy(
                rows_hbm.at[pl.ds(tbase + o * _SRD + s * CH, CH)],
                idxr.at[s], semi).wait()

    _rd(0, mrbuf0, semr0)
    _rd(1, mrbuf1, semr1)
    _rdidx(0, idxr0, semi0)
    _rdidx(1, idxr1, semi1)

    def sstep(k, carry):
        for b, (buf, idxr, semr, sema, semi) in enumerate(
                ((mrbuf0, idxr0, semr0, sema0, semi0),
                 (mrbuf1, idxr1, semr1, sema1, semi1))):
            o = 2 * k + b
            _widx(o, idxr, semi)
            pltpu.make_async_copy(
                mr2_hbm.at[c, pl.ds(tbase + o * _SRD, _SRD)], buf, semr).wait()
            for s in range(_SUB):
                pltpu.async_copy(
                    buf.at[pl.ds(s * CH, CH)],
                    acc.at[idxr.at[s]], sema, add=True)
            for s in range(_SUB):
                pltpu.make_async_copy(
                    buf.at[pl.ds(s * CH, CH)],
                    acc.at[idxr.at[s]], sema).wait()

            @pl.when(o + 2 < _S_OUTER)
            def _():
                _rd(o + 2, buf, semr)
                _rdidx(o + 2, idxr, semi)
        return carry

    lax.fori_loop(0, _S_OUTER // 2, sstep, 0)
    plsc.subcore_barrier()

    def wstep(k, carry):
        off = t * (N // 16) + k * 125
        pltpu.sync_copy(acc.at[pl.ds(off, 125)], mrbuf0.at[pl.ds(0, 125)])
        pltpu.sync_copy(mrbuf0.at[pl.ds(0, 125)],
                        s2_hbm.at[c, pl.ds(off, 125)])
        return carry

    lax.fori_loop(0, (N // 16) // 125, wstep, 0)


def kernel(x_batch, edge_index, edge_attr, params):
    row = edge_index[0]
    col = edge_index[1]
    deg = jnp.zeros((N, 1), _F32).at[row, 0].add(1.0)

    pad = E_PAD - E
    row_g = jnp.concatenate([row, jnp.zeros((pad,), jnp.int32)])
    col_g = jnp.concatenate([col, jnp.zeros((pad,), jnp.int32)])
    row_s = jnp.concatenate([row, jnp.full((pad,), N, jnp.int32)])
    ea_pad = jnp.concatenate([edge_attr, jnp.zeros((pad, 2), _F32)])
    zeros128 = jnp.zeros((CH, 32), _F32)

    W1 = params["W1"]
    W1aT = jnp.swapaxes(W1[:, :, :D], 1, 2)
    W1bT = jnp.swapaxes(W1[:, :, D:2 * D], 1, 2)
    W1cT = jnp.swapaxes(W1[:, :, 2 * D:], 1, 2)
    W2T = jnp.swapaxes(params["W2"], 1, 2)
    W3 = params["W3"]
    W3hT = jnp.swapaxes(W3[:, :, :D], 1, 2)
    W3aT = jnp.swapaxes(W3[:, :, D:], 1, 2)
    W1aT_nxt = jnp.roll(W1aT, -1, axis=0)
    W1bT_nxt = jnp.roll(W1bT, -1, axis=0)
    row2 = lambda a: a.reshape(1, -1)
    WeT = params["We"].T
    be = row2(params["be"])
    Wd1T = params["Wd1"].T
    bd1 = row2(params["bd1"])
    Wd2T = params["Wd2"].T
    bd2 = row2(params["bd2"])

    b1 = params["b1"][:, None, :]
    g1 = params["g1"][:, None, :]
    bb1 = params["bb1"][:, None, :]
    b2 = params["b2"][:, None, :]
    b3 = params["b3"][:, None, :]
    g2 = params["g2"][:, None, :]
    bb2 = params["bb2"][:, None, :]

    layer_xs = (W1cT, b1, g1, bb1, W2T, b2, W3hT, W3aT, b3, g2, bb2,
                W1aT_nxt, W1bT_nxt)

    def single(x):
        h, pq = _embed(x, WeT, be, W1aT[0], W1bT[0])

        def body(carry, lp):
            h, pq = carry
            (w1cT, b1l, g1l, bb1l, w2T, b2l, w3hT, w3aT, b3l, g2l, bb2l,
             w1aTn, w1bTn) = lp
            gr, gc = _sc_gather(pq, row_g, col_g)
            mr2 = _edge_stage(gr, gc, ea_pad, w1cT, b1l, g1l, bb1l)
            s2 = _sc_scatter(mr2, row_s, zeros128)
            h, pq = _node_stage(h, s2, deg, w2T, b2l, w3hT, w3aT, b3l,
                                g2l, bb2l, w1aTn, w1bTn)
            return (h, pq), None

        (h, _), _ = lax.scan(body, (h, pq), layer_xs)
        return _decode(h, Wd1T, bd1, Wd2T, bd2)

    return jnp.stack([single(x_batch[0]), single(x_batch[1])], axis=0)


# speedup vs baseline: 19.2479x; 1.1004x over previous
import functools

import jax
import jax.numpy as jnp
from jax import lax
from jax.experimental import pallas as pl
from jax.experimental.pallas import tpu as pltpu
from jax.experimental.pallas import tpu_sc as plsc

N = 50000
E = 800000
D = 64
L = 25
B = 2

NBLK = 2000
EBLK = 8192
CH = 128
E_PAD = 802816
ACC_ROWS = 51200

_F32 = jnp.float32


def _ln(x, g, b, eps=1e-5):
    mu = jnp.mean(x, axis=-1, keepdims=True)
    var = jnp.mean((x - mu) * (x - mu), axis=-1, keepdims=True)
    return (x - mu) / jnp.sqrt(var + eps) * g + b


def _mm(a, b):
    return jax.lax.dot_general(a, b, (((1,), (0,)), ((), ())),
                               preferred_element_type=_F32)


def _embed_body(x_ref, WeT, be, W1aT, W1bT, h_out, pq_out):
    h = _mm(x_ref[...], WeT[...]) + be[...]
    h_out[...] = h
    pq_out[...] = jnp.concatenate([_mm(h, W1aT[...]), _mm(h, W1bT[...])],
                                  axis=1)


def _embed(x, WeT, be, W1aT0, W1bT0):
    grid = (N // NBLK,)
    bs_n = lambda c: pl.BlockSpec((NBLK, c), lambda i: (i, 0))
    full = lambda a: pl.BlockSpec(a.shape, lambda i: (0,) * a.ndim)
    return pl.pallas_call(
        _embed_body,
        grid=grid,
        in_specs=[bs_n(3), full(WeT), full(be), full(W1aT0), full(W1bT0)],
        out_specs=[bs_n(D), bs_n(2 * D)],
        out_shape=[jax.ShapeDtypeStruct((N, D), _F32),
                   jax.ShapeDtypeStruct((N, 2 * D), _F32)],
    )(x, WeT, be, W1aT0, W1bT0)


def _edge_body(gr_ref, gc_ref, ea_ref, W1cT, b1, g1, bb1, mr2_out):
    m = (gr_ref[...][:, :D] + gc_ref[...][:, D:]
         + _mm(ea_ref[...], W1cT[...]) + b1[...])
    mr = jax.nn.relu(_ln(m, g1[...], bb1[...]))
    mr2_out[...] = jnp.stack([mr[:, :32], mr[:, 32:]], axis=0)


def _edge_stage(gr, gc, ea, W1cT, b1, g1, bb1):
    grid = (E_PAD // EBLK,)
    bs_e = lambda c: pl.BlockSpec((EBLK, c), lambda i: (i, 0))
    full = lambda a: pl.BlockSpec(a.shape, lambda i: (0,) * a.ndim)
    return pl.pallas_call(
        _edge_body,
        grid=grid,
        in_specs=[bs_e(2 * D), bs_e(2 * D), bs_e(2), full(W1cT), full(b1),
                  full(g1), full(bb1)],
        out_specs=pl.BlockSpec((2, EBLK, 32), lambda i: (0, i, 0)),
        out_shape=jax.ShapeDtypeStruct((2, E_PAD, 32), _F32),
    )(gr, gc, ea, W1cT, b1, g1, bb1)


def _node_body(h_ref, s2_ref, deg_ref, W2T, b2, W3hT, W3aT, b3, g2, bb2,
               W1aT, W1bT, h_out, pq_out):
    s2 = s2_ref[...]
    S = jnp.concatenate([s2[0], s2[1]], axis=1)
    aggr = _mm(S, W2T[...]) + deg_ref[...] * b2[...]
    h = h_ref[...]
    u = _mm(h, W3hT[...]) + _mm(aggr, W3aT[...]) + b3[...]
    u = jax.nn.relu(_ln(u, g2[...], bb2[...]))
    hn = h + u
    h_out[...] = hn
    pq_out[...] = jnp.concatenate([_mm(hn, W1aT[...]), _mm(hn, W1bT[...])],
                                  axis=1)


def _node_stage(h, s2, deg, W2T, b2, W3hT, W3aT, b3, g2, bb2, W1aT, W1bT):
    grid = (N // NBLK,)
    bs_n = lambda c: pl.BlockSpec((NBLK, c), lambda i: (i, 0))
    full = lambda a: pl.BlockSpec(a.shape, lambda i: (0,) * a.ndim)
    return pl.pallas_call(
        _node_body,
        grid=grid,
        in_specs=[bs_n(D), pl.BlockSpec((2, NBLK, 32), lambda i: (0, i, 0)),
                  bs_n(1), full(W2T), full(b2), full(W3hT), full(W3aT),
                  full(b3), full(g2), full(bb2), full(W1aT), full(W1bT)],
        out_specs=[bs_n(D), bs_n(2 * D)],
        out_shape=[jax.ShapeDtypeStruct((N, D), _F32),
                   jax.ShapeDtypeStruct((N, 2 * D), _F32)],
    )(h, s2, deg, W2T, b2, W3hT, W3aT, b3, g2, bb2, W1aT, W1bT)


def _dec_body(h_ref, Wd1T, bd1, Wd2T, bd2, out_ref):
    hid = jax.nn.relu(_mm(h_ref[...], Wd1T[...]) + bd1[...])
    out_ref[...] = _mm(hid, Wd2T[...]) + bd2[...]


def _decode(h, Wd1T, bd1, Wd2T, bd2):
    grid = (N // NBLK,)
    bs_n = lambda c: pl.BlockSpec((NBLK, c), lambda i: (i, 0))
    full = lambda a: pl.BlockSpec(a.shape, lambda i: (0,) * a.ndim)
    return pl.pallas_call(
        _dec_body,
        grid=grid,
        in_specs=[bs_n(D), full(Wd1T), full(bd1), full(Wd2T), full(bd2)],
        out_specs=bs_n(2),
        out_shape=jax.ShapeDtypeStruct((N, 2), _F32),
    )(h, Wd1T, bd1, Wd2T, bd2)


_SC_MESH = plsc.VectorSubcoreMesh(core_axis_name="c", subcore_axis_name="s",
                                  num_cores=2, num_subcores=16)
_SC_PARAMS_TILED = pltpu.CompilerParams(use_tc_tiling_on_sc=True)
_SC_PARAMS = pltpu.CompilerParams(use_tc_tiling_on_sc=False)

_E_TILE = E_PAD // 32
_GC = 392
_G_CHUNKS = _E_TILE // _GC
_S_CHUNKS = E_PAD // (16 * CH)
_SRD = 256
_S_OUTER = E_PAD // (16 * _SRD)


def _gather_pass(tab_hbm, idx_hbm, out_hbm, idx_all, buf0, buf1, sem0, sem1,
                 tbase):
    pltpu.sync_copy(idx_hbm.at[pl.ds(tbase, _E_TILE)], idx_all)

    def _g(j, buf, sem):
        return pltpu.async_copy(
            tab_hbm.at[idx_all.at[pl.ds(j * _GC, _GC)]], buf, sem)

    _g(0, buf0, sem0)
    _g(1, buf1, sem1)

    def step(k, carry):
        for b, (buf, sem) in enumerate(((buf0, sem0), (buf1, sem1))):
            j = 2 * k + b
            pltpu.make_async_copy(
                tab_hbm.at[idx_all.at[pl.ds(j * _GC, _GC)]], buf, sem).wait()
            pltpu.sync_copy(buf, out_hbm.at[pl.ds(tbase + j * _GC, _GC)])

            @pl.when(j + 2 < _G_CHUNKS)
            def _():
                _g(j + 2, buf, sem)
        return carry

    lax.fori_loop(0, _G_CHUNKS // 2, step, 0)


@functools.partial(
    pl.kernel,
    out_type=[jax.ShapeDtypeStruct((E_PAD, 2 * D), _F32)] * 2,
    mesh=_SC_MESH,
    scratch_types=[
        pltpu.VMEM((_E_TILE,), jnp.int32),
        pltpu.VMEM((_GC, 2 * D), _F32),
        pltpu.VMEM((_GC, 2 * D), _F32),
        pltpu.SemaphoreType.DMA,
        pltpu.SemaphoreType.DMA,
    ],
    compiler_params=_SC_PARAMS_TILED,
)
def _sc_gather(pq_hbm, row_hbm, col_hbm, gr_hbm, gc_hbm,
               idx_all, buf0, buf1, sem0, sem1):
    wid = lax.axis_index("s") * 2 + lax.axis_index("c")
    tbase = wid * _E_TILE
    _gather_pass(pq_hbm, row_hbm, gr_hbm, idx_all, buf0, buf1, sem0, sem1,
                 tbase)
    _gather_pass(pq_hbm, col_hbm, gc_hbm, idx_all, buf0, buf1, sem0, sem1,
                 tbase)


_WRITE_CHUNK = 625


@functools.partial(
    pl.kernel,
    out_type=jax.ShapeDtypeStruct((2, N, 32), _F32),
    mesh=_SC_MESH,
    scratch_types=[
        pltpu.VMEM_SHARED((ACC_ROWS, 32), _F32),
        pltpu.VMEM((_SRD // CH, CH), jnp.int32),
        pltpu.VMEM((_SRD // CH, CH), jnp.int32),
        pltpu.VMEM((_SRD, 32), _F32),
        pltpu.VMEM((_SRD, 32), _F32),
        pltpu.SemaphoreType.DMA,
        pltpu.SemaphoreType.DMA,
        pltpu.SemaphoreType.DMA,
        pltpu.SemaphoreType.DMA,
        pltpu.SemaphoreType.DMA,
        pltpu.SemaphoreType.DMA,
    ],
    compiler_params=_SC_PARAMS,
)
def _sc_scatter(mr2_hbm, rows_hbm, zeros_hbm, s2_hbm,
                acc, idxr0, idxr1, mrbuf0, mrbuf1,
                semr0, semr1, sema0, sema1, semi0, semi1):
    c = lax.axis_index("c")
    t = lax.axis_index("s")
    tbase = t * _S_OUTER * _SRD
    _SUB = _SRD // CH

    pltpu.sync_copy(zeros_hbm, mrbuf0.at[pl.ds(0, CH)])

    def zstep(j, carry):
        pltpu.sync_copy(mrbuf0.at[pl.ds(0, CH)],
                        acc.at[pl.ds((t * 25 + j) * CH, CH)])
        return carry

    lax.fori_loop(0, ACC_ROWS // (16 * CH), zstep, 0)
    plsc.subcore_barrier()

    def _rd(o, buf, sem):
        return pltpu.async_copy(
            mr2_hbm.at[c, pl.ds(tbase + o * _SRD, _SRD)], buf, sem)

    def _rdidx(o, idxr, semi):
        for s in range(_SUB):
            pltpu.async_copy(
                rows_hbm.at[pl.ds(tbase + o * _SRD + s * CH, CH)],
                idxr.at[s], semi)

    def _widx(o, idxr, semi):
        for s in range(_SUB):
            pltpu.make_async_copy(
                rows_hbm.at[pl.ds(tbase + o * _SRD + s * CH, CH)],
                idxr.at[s], semi).wait()

    _rd(0, mrbuf0, semr0)
    _rd(1, mrbuf1, semr1)
    _rdidx(0, idxr0, semi0)
    _rdidx(1, idxr1, semi1)

    def sstep(k, carry):
        for b, (buf, idxr, semr, sema, semi) in enumerate(
                ((mrbuf0, idxr0, semr0, sema0, semi0),
                 (mrbuf1, idxr1, semr1, sema1, semi1))):
            o = 2 * k + b
            _widx(o, idxr, semi)
            pltpu.make_async_copy(
                mr2_hbm.at[c, pl.ds(tbase + o * _SRD, _SRD)], buf, semr).wait()
            for s in range(_SUB):
                pltpu.async_copy(
                    buf.at[pl.ds(s * CH, CH)],
                    acc.at[idxr.at[s]], sema, add=True)
            for s in range(_SUB):
                pltpu.make_async_copy(
                    buf.at[pl.ds(s * CH, CH)],
                    acc.at[idxr.at[s]], sema).wait()

            @pl.when(o + 2 < _S_OUTER)
            def _():
                _rd(o + 2, buf, semr)
                _rdidx(o + 2, idxr, semi)
        return carry

    lax.fori_loop(0, _S_OUTER // 2, sstep, 0)
    plsc.subcore_barrier()

    def wstep(k, carry):
        off = t * (N // 16) + k * 125
        pltpu.sync_copy(acc.at[pl.ds(off, 125)], mrbuf0.at[pl.ds(0, 125)])
        pltpu.sync_copy(mrbuf0.at[pl.ds(0, 125)],
                        s2_hbm.at[c, pl.ds(off, 125)])
        return carry

    lax.fori_loop(0, (N // 16) // 125, wstep, 0)


def kernel(x_batch, edge_index, edge_attr, params):
    row = edge_index[0]
    col = edge_index[1]
    deg = jnp.zeros((N, 1), _F32).at[row, 0].add(1.0)

    pad = E_PAD - E
    row_g = jnp.concatenate([row, jnp.zeros((pad,), jnp.int32)])
    col_g = jnp.concatenate([col, jnp.zeros((pad,), jnp.int32)])
    row_s = jnp.concatenate([row, jnp.full((pad,), N, jnp.int32)])
    ea_pad = jnp.concatenate([edge_attr, jnp.zeros((pad, 2), _F32)])
    zeros128 = jnp.zeros((CH, 32), _F32)

    W1 = params["W1"]
    W1aT = jnp.swapaxes(W1[:, :, :D], 1, 2)
    W1bT = jnp.swapaxes(W1[:, :, D:2 * D], 1, 2)
    W1cT = jnp.swapaxes(W1[:, :, 2 * D:], 1, 2)
    W2T = jnp.swapaxes(params["W2"], 1, 2)
    W3 = params["W3"]
    W3hT = jnp.swapaxes(W3[:, :, :D], 1, 2)
    W3aT = jnp.swapaxes(W3[:, :, D:], 1, 2)
    W1aT_nxt = jnp.roll(W1aT, -1, axis=0)
    W1bT_nxt = jnp.roll(W1bT, -1, axis=0)
    row2 = lambda a: a.reshape(1, -1)
    WeT = params["We"].T
    be = row2(params["be"])
    Wd1T = params["Wd1"].T
    bd1 = row2(params["bd1"])
    Wd2T = params["Wd2"].T
    bd2 = row2(params["bd2"])

    b1 = params["b1"][:, None, :]
    g1 = params["g1"][:, None, :]
    bb1 = params["bb1"][:, None, :]
    b2 = params["b2"][:, None, :]
    b3 = params["b3"][:, None, :]
    g2 = params["g2"][:, None, :]
    bb2 = params["bb2"][:, None, :]

    layer_xs = (W1cT, b1, g1, bb1, W2T, b2, W3hT, W3aT, b3, g2, bb2,
                W1aT_nxt, W1bT_nxt)

    h0, pq0 = _embed(x_batch[0], WeT, be, W1aT[0], W1bT[0])
    h1, pq1 = _embed(x_batch[1], WeT, be, W1aT[0], W1bT[0])

    def body(carry, lp):
        h0, pq0, h1, pq1 = carry
        (w1cT, b1l, g1l, bb1l, w2T, b2l, w3hT, w3aT, b3l, g2l, bb2l,
         w1aTn, w1bTn) = lp
        gr0, gc0 = _sc_gather(pq0, row_g, col_g)
        gr1, gc1 = _sc_gather(pq1, row_g, col_g)
        mr20 = _edge_stage(gr0, gc0, ea_pad, w1cT, b1l, g1l, bb1l)
        mr21 = _edge_stage(gr1, gc1, ea_pad, w1cT, b1l, g1l, bb1l)
        s20 = _sc_scatter(mr20, row_s, zeros128)
        s21 = _sc_scatter(mr21, row_s, zeros128)
        h0, pq0 = _node_stage(h0, s20, deg, w2T, b2l, w3hT, w3aT, b3l,
                              g2l, bb2l, w1aTn, w1bTn)
        h1, pq1 = _node_stage(h1, s21, deg, w2T, b2l, w3hT, w3aT, b3l,
                              g2l, bb2l, w1aTn, w1bTn)
        return (h0, pq0, h1, pq1), None

    (h0, _, h1, _), _ = lax.scan(body, (h0, pq0, h1, pq1), layer_xs)
    return jnp.stack([_decode(h0, Wd1T, bd1, Wd2T, bd2),
                      _decode(h1, Wd1T, bd1, Wd2T, bd2)], axis=0)


# speedup vs baseline: 19.2833x; 1.0018x over previous
import functools

import jax
import jax.numpy as jnp
from jax import lax
from jax.experimental import pallas as pl
from jax.experimental.pallas import tpu as pltpu
from jax.experimental.pallas import tpu_sc as plsc

N = 50000
E = 800000
D = 64
L = 25
B = 2

NBLK = 2000
EBLK = 8192
CH = 128
E_PAD = 802816
ACC_ROWS = 51200

_F32 = jnp.float32


def _ln(x, g, b, eps=1e-5):
    mu = jnp.mean(x, axis=-1, keepdims=True)
    var = jnp.mean((x - mu) * (x - mu), axis=-1, keepdims=True)
    return (x - mu) / jnp.sqrt(var + eps) * g + b


def _mm(a, b):
    return jax.lax.dot_general(a, b, (((1,), (0,)), ((), ())),
                               preferred_element_type=_F32)


def _embed_body(x_ref, WeT, be, W1aT, W1bT, h_out, pq_out):
    h = _mm(x_ref[...], WeT[...]) + be[...]
    h_out[...] = h
    pq_out[...] = jnp.concatenate([_mm(h, W1aT[...]), _mm(h, W1bT[...])],
                                  axis=1)


def _embed(x, WeT, be, W1aT0, W1bT0):
    grid = (N // NBLK,)
    bs_n = lambda c: pl.BlockSpec((NBLK, c), lambda i: (i, 0))
    full = lambda a: pl.BlockSpec(a.shape, lambda i: (0,) * a.ndim)
    return pl.pallas_call(
        _embed_body,
        grid=grid,
        in_specs=[bs_n(3), full(WeT), full(be), full(W1aT0), full(W1bT0)],
        out_specs=[bs_n(D), bs_n(2 * D)],
        out_shape=[jax.ShapeDtypeStruct((N, D), _F32),
                   jax.ShapeDtypeStruct((N, 2 * D), _F32)],
    )(x, WeT, be, W1aT0, W1bT0)


def _edge_body(gr_ref, gc_ref, ea_ref, W1cT, b1, g1, bb1, mr2_out):
    m = (gr_ref[...][:, :D] + gc_ref[...][:, D:]
         + _mm(ea_ref[...], W1cT[...]) + b1[...])
    mr = jax.nn.relu(_ln(m, g1[...], bb1[...]))
    mr2_out[...] = jnp.stack([mr[:, :32], mr[:, 32:]], axis=0)


def _edge_stage(gr, gc, ea, W1cT, b1, g1, bb1):
    grid = (E_PAD // EBLK,)
    bs_e = lambda c: pl.BlockSpec((EBLK, c), lambda i: (i, 0))
    full = lambda a: pl.BlockSpec(a.shape, lambda i: (0,) * a.ndim)
    return pl.pallas_call(
        _edge_body,
        grid=grid,
        in_specs=[bs_e(2 * D), bs_e(2 * D), bs_e(2), full(W1cT), full(b1),
                  full(g1), full(bb1)],
        out_specs=pl.BlockSpec((2, EBLK, 32), lambda i: (0, i, 0)),
        out_shape=jax.ShapeDtypeStruct((2, E_PAD, 32), _F32),
    )(gr, gc, ea, W1cT, b1, g1, bb1)


def _node_body(h_ref, s2_ref, deg_ref, W2T, b2, W3hT, W3aT, b3, g2, bb2,
               W1aT, W1bT, h_out, pq_out):
    s2 = s2_ref[...]
    S = jnp.concatenate([s2[0], s2[1]], axis=1)
    aggr = _mm(S, W2T[...]) + deg_ref[...] * b2[...]
    h = h_ref[...]
    u = _mm(h, W3hT[...]) + _mm(aggr, W3aT[...]) + b3[...]
    u = jax.nn.relu(_ln(u, g2[...], bb2[...]))
    hn = h + u
    h_out[...] = hn
    pq_out[...] = jnp.concatenate([_mm(hn, W1aT[...]), _mm(hn, W1bT[...])],
                                  axis=1)


def _node_stage(h, s2, deg, W2T, b2, W3hT, W3aT, b3, g2, bb2, W1aT, W1bT):
    grid = (N // NBLK,)
    bs_n = lambda c: pl.BlockSpec((NBLK, c), lambda i: (i, 0))
    full = lambda a: pl.BlockSpec(a.shape, lambda i: (0,) * a.ndim)
    return pl.pallas_call(
        _node_body,
        grid=grid,
        in_specs=[bs_n(D), pl.BlockSpec((2, NBLK, 32), lambda i: (0, i, 0)),
                  bs_n(1), full(W2T), full(b2), full(W3hT), full(W3aT),
                  full(b3), full(g2), full(bb2), full(W1aT), full(W1bT)],
        out_specs=[bs_n(D), bs_n(2 * D)],
        out_shape=[jax.ShapeDtypeStruct((N, D), _F32),
                   jax.ShapeDtypeStruct((N, 2 * D), _F32)],
    )(h, s2, deg, W2T, b2, W3hT, W3aT, b3, g2, bb2, W1aT, W1bT)


def _dec_body(h_ref, Wd1T, bd1, Wd2T, bd2, out_ref):
    hid = jax.nn.relu(_mm(h_ref[...], Wd1T[...]) + bd1[...])
    out_ref[...] = _mm(hid, Wd2T[...]) + bd2[...]


def _decode(h, Wd1T, bd1, Wd2T, bd2):
    grid = (N // NBLK,)
    bs_n = lambda c: pl.BlockSpec((NBLK, c), lambda i: (i, 0))
    full = lambda a: pl.BlockSpec(a.shape, lambda i: (0,) * a.ndim)
    return pl.pallas_call(
        _dec_body,
        grid=grid,
        in_specs=[bs_n(D), full(Wd1T), full(bd1), full(Wd2T), full(bd2)],
        out_specs=bs_n(2),
        out_shape=jax.ShapeDtypeStruct((N, 2), _F32),
    )(h, Wd1T, bd1, Wd2T, bd2)


_SC_MESH = plsc.VectorSubcoreMesh(core_axis_name="c", subcore_axis_name="s",
                                  num_cores=2, num_subcores=16)
_SC_PARAMS_TILED = pltpu.CompilerParams(use_tc_tiling_on_sc=True)
_SC_PARAMS = pltpu.CompilerParams(use_tc_tiling_on_sc=False)

_E_TILE = E_PAD // 32
_GC = 392
_G_CHUNKS = _E_TILE // _GC
_S_CHUNKS = E_PAD // (16 * CH)
_SRD = 256
_S_OUTER = E_PAD // (16 * _SRD)


def _gather_pass(tab_hbm, idx_hbm, out_hbm, idx_all, buf0, buf1, sem0, sem1,
                 tbase):
    pltpu.sync_copy(idx_hbm.at[pl.ds(tbase, _E_TILE)], idx_all)

    def _g(j, buf, sem):
        return pltpu.async_copy(
            tab_hbm.at[idx_all.at[pl.ds(j * _GC, _GC)]], buf, sem)

    _g(0, buf0, sem0)
    _g(1, buf1, sem1)

    def step(k, carry):
        for b, (buf, sem) in enumerate(((buf0, sem0), (buf1, sem1))):
            j = 2 * k + b
            pltpu.make_async_copy(
                tab_hbm.at[idx_all.at[pl.ds(j * _GC, _GC)]], buf, sem).wait()
            pltpu.sync_copy(buf, out_hbm.at[pl.ds(tbase + j * _GC, _GC)])

            @pl.when(j + 2 < _G_CHUNKS)
            def _():
                _g(j + 2, buf, sem)
        return carry

    lax.fori_loop(0, _G_CHUNKS // 2, step, 0)


@functools.partial(
    pl.kernel,
    out_type=[jax.ShapeDtypeStruct((E_PAD, 2 * D), _F32)] * 2,
    mesh=_SC_MESH,
    scratch_types=[
        pltpu.VMEM((_E_TILE,), jnp.int32),
        pltpu.VMEM((_GC, 2 * D), _F32),
        pltpu.VMEM((_GC, 2 * D), _F32),
        pltpu.SemaphoreType.DMA,
        pltpu.SemaphoreType.DMA,
    ],
    compiler_params=_SC_PARAMS_TILED,
)
def _sc_gather(pq_hbm, row_hbm, col_hbm, gr_hbm, gc_hbm,
               idx_all, buf0, buf1, sem0, sem1):
    wid = lax.axis_index("s") * 2 + lax.axis_index("c")
    tbase = wid * _E_TILE
    _gather_pass(pq_hbm, row_hbm, gr_hbm, idx_all, buf0, buf1, sem0, sem1,
                 tbase)
    _gather_pass(pq_hbm, col_hbm, gc_hbm, idx_all, buf0, buf1, sem0, sem1,
                 tbase)


@functools.partial(
    pl.kernel,
    out_type=jax.ShapeDtypeStruct((2, N, 32), _F32),
    mesh=_SC_MESH,
    scratch_types=[
        pltpu.VMEM_SHARED((ACC_ROWS, 32), _F32),
        pltpu.VMEM((_SRD // CH, CH), jnp.int32),
        pltpu.VMEM((_SRD // CH, CH), jnp.int32),
        pltpu.VMEM((_SRD, 32), _F32),
        pltpu.VMEM((_SRD, 32), _F32),
        pltpu.SemaphoreType.DMA,
        pltpu.SemaphoreType.DMA,
        pltpu.SemaphoreType.DMA,
        pltpu.SemaphoreType.DMA,
        pltpu.SemaphoreType.DMA,
        pltpu.SemaphoreType.DMA,
    ],
    compiler_params=_SC_PARAMS,
)
def _sc_scatter(mr2_hbm, rows_hbm, zeros_hbm, s2_hbm,
                acc, idxr0, idxr1, mrbuf0, mrbuf1,
                semr0, semr1, sema0, sema1, semi0, semi1):
    c = lax.axis_index("c")
    t = lax.axis_index("s")
    tbase = t * _S_OUTER * _SRD
    _SUB = _SRD // CH

    pltpu.sync_copy(zeros_hbm, mrbuf0.at[pl.ds(0, CH)])

    def zstep(j, carry):
        pltpu.sync_copy(mrbuf0.at[pl.ds(0, CH)],
                        acc.at[pl.ds((t * 25 + j) * CH, CH)])
        return carry

    lax.fori_loop(0, ACC_ROWS // (16 * CH), zstep, 0)
    plsc.subcore_barrier()

    def _rd(o, buf, sem):
        return pltpu.async_copy(
            mr2_hbm.at[c, pl.ds(tbase + o * _SRD, _SRD)], buf, sem)

    def _rdidx(o, idxr, semi):
        for s in range(_SUB):
            pltpu.async_copy(
                rows_hbm.at[pl.ds(tbase + o * _SRD + s * CH, CH)],
                idxr.at[s], semi)

    def _widx(o, idxr, semi):
        for s in range(_SUB):
            pltpu.make_async_copy(
                rows_hbm.at[pl.ds(tbase + o * _SRD + s * CH, CH)],
                idxr.at[s], semi).wait()

    _rd(0, mrbuf0, semr0)
    _rd(1, mrbuf1, semr1)
    _rdidx(0, idxr0, semi0)
    _rdidx(1, idxr1, semi1)

    def sstep(k, carry):
        for b, (buf, idxr, semr, sema, semi) in enumerate(
                ((mrbuf0, idxr0, semr0, sema0, semi0),
                 (mrbuf1, idxr1, semr1, sema1, semi1))):
            o = 2 * k + b
            _widx(o, idxr, semi)
            pltpu.make_async_copy(
                mr2_hbm.at[c, pl.ds(tbase + o * _SRD, _SRD)], buf, semr).wait()
            for s in range(_SUB):
                pltpu.async_copy(
                    buf.at[pl.ds(s * CH, CH)],
                    acc.at[idxr.at[s]], sema, add=True)
            for s in range(_SUB):
                pltpu.make_async_copy(
                    buf.at[pl.ds(s * CH, CH)],
                    acc.at[idxr.at[s]], sema).wait()

            @pl.when(o + 2 < _S_OUTER)
            def _():
                _rd(o + 2, buf, semr)
                _rdidx(o + 2, idxr, semi)
        return carry

    lax.fori_loop(0, _S_OUTER // 2, sstep, 0)
    plsc.subcore_barrier()

    def wstep(k, carry):
        off = t * (N // 16) + k * 125
        pltpu.sync_copy(acc.at[pl.ds(off, 125)], mrbuf0.at[pl.ds(0, 125)])
        pltpu.sync_copy(mrbuf0.at[pl.ds(0, 125)],
                        s2_hbm.at[c, pl.ds(off, 125)])
        return carry

    lax.fori_loop(0, (N // 16) // 125, wstep, 0)


def kernel(x_batch, edge_index, edge_attr, params):
    row = edge_index[0]
    col = edge_index[1]
    deg = jnp.zeros((N, 1), _F32).at[row, 0].add(1.0)

    pad = E_PAD - E
    row_g = jnp.concatenate([row, jnp.zeros((pad,), jnp.int32)])
    col_g = jnp.concatenate([col, jnp.zeros((pad,), jnp.int32)])
    row_s = jnp.concatenate([row, jnp.full((pad,), N, jnp.int32)])
    ea_pad = jnp.concatenate([edge_attr, jnp.zeros((pad, 2), _F32)])
    zeros128 = jnp.zeros((CH, 32), _F32)

    W1 = params["W1"]
    W1aT = jnp.swapaxes(W1[:, :, :D], 1, 2)
    W1bT = jnp.swapaxes(W1[:, :, D:2 * D], 1, 2)
    W1cT = jnp.swapaxes(W1[:, :, 2 * D:], 1, 2)
    W2T = jnp.swapaxes(params["W2"], 1, 2)
    W3 = params["W3"]
    W3hT = jnp.swapaxes(W3[:, :, :D], 1, 2)
    W3aT = jnp.swapaxes(W3[:, :, D:], 1, 2)
    W1aT_nxt = jnp.roll(W1aT, -1, axis=0)
    W1bT_nxt = jnp.roll(W1bT, -1, axis=0)
    row2 = lambda a: a.reshape(1, -1)
    WeT = params["We"].T
    be = row2(params["be"])
    Wd1T = params["Wd1"].T
    bd1 = row2(params["bd1"])
    Wd2T = params["Wd2"].T
    bd2 = row2(params["bd2"])

    b1 = params["b1"][:, None, :]
    g1 = params["g1"][:, None, :]
    bb1 = params["bb1"][:, None, :]
    b2 = params["b2"][:, None, :]
    b3 = params["b3"][:, None, :]
    g2 = params["g2"][:, None, :]
    bb2 = params["bb2"][:, None, :]

    layer_xs = (W1cT, b1, g1, bb1, W2T, b2, W3hT, W3aT, b3, g2, bb2,
                W1aT_nxt, W1bT_nxt)

    h0, pq0 = _embed(x_batch[0], WeT, be, W1aT[0], W1bT[0])
    h1, pq1 = _embed(x_batch[1], WeT, be, W1aT[0], W1bT[0])

    def body(carry, lp):
        h0, pq0, h1, pq1 = carry
        (w1cT, b1l, g1l, bb1l, w2T, b2l, w3hT, w3aT, b3l, g2l, bb2l,
         w1aTn, w1bTn) = lp
        gr0, gc0 = _sc_gather(pq0, row_g, col_g)
        gr1, gc1 = _sc_gather(pq1, row_g, col_g)
        mr20 = _edge_stage(gr0, gc0, ea_pad, w1cT, b1l, g1l, bb1l)
        mr21 = _edge_stage(gr1, gc1, ea_pad, w1cT, b1l, g1l, bb1l)
        s20 = _sc_scatter(mr20, row_s, zeros128)
        s21 = _sc_scatter(mr21, row_s, zeros128)
        h0, pq0 = _node_stage(h0, s20, deg, w2T, b2l, w3hT, w3aT, b3l,
                              g2l, bb2l, w1aTn, w1bTn)
        h1, pq1 = _node_stage(h1, s21, deg, w2T, b2l, w3hT, w3aT, b3l,
                              g2l, bb2l, w1aTn, w1bTn)
        return (h0, pq0, h1, pq1), None

    (h0, _, h1, _), _ = lax.scan(body, (h0, pq0, h1, pq1), layer_xs)
    return jnp.stack([_decode(h0, Wd1T, bd1, Wd2T, bd2),
                      _decode(h1, Wd1T, bd1, Wd2T, bd2)], axis=0)
